# baseline (device time: 23437 ns/iter reference)
import numpy as np
import jax
import jax.numpy as jnp
from jax import lax
from jax.experimental import pallas as pl
from jax.experimental.pallas import tpu as pltpu

N_DEV = 4
B, SQ, D = 2, 256, 768
HQ_LOCAL, DH = 4, 64
DLOC = HQ_LOCAL * DH
CHUNKS = [(0, 128), (128, 128), (256, 96), (352, 96), (448, 64)]


def _rope_tables():
    inv = 1.0 / (10000.0 ** (np.arange(0, DH, 2) / DH))
    pos = np.arange(SQ)[:, None] * inv[None, :]
    cos = np.repeat(np.cos(pos), 2, axis=-1).astype(np.float32)
    sin = np.repeat(np.sin(pos), 2, axis=-1).astype(np.float32)
    return np.tile(cos, (B, HQ_LOCAL)), np.tile(sin, (B, HQ_LOCAL))


def kernel(x, Wq, Wk, Wv, Wo):
    bf16 = jnp.bfloat16
    n_chunk = len(CHUNKS)

    def body(x_ref, wqkv_ref, wo_ref, cos_ref, sin_ref,
             out_ref, send_buf, recv_buf, send_sems, recv_sems):
        my_pos = lax.axis_index("i")
        p_a = my_pos ^ 1
        p_b = 3 - my_pos

        barrier_sem = pltpu.get_barrier_semaphore()
        for nbr in (p_a, p_b):
            pl.semaphore_signal(
                barrier_sem, inc=1,
                device_id=(nbr,), device_id_type=pl.DeviceIdType.MESH,
            )
        pl.semaphore_wait(barrier_sem, 2)

        def exchange(stage, c):
            start, size = CHUNKS[c]
            first, second = (p_a, p_b) if c % 2 == 0 else (p_b, p_a)
            partner = first if stage == 0 else second
            return pltpu.make_async_remote_copy(
                src_ref=send_buf.at[start:start + size],
                dst_ref=recv_buf.at[stage, start:start + size],
                send_sem=send_sems.at[stage, c],
                recv_sem=recv_sems.at[stage, c],
                device_id=(partner,),
                device_id_type=pl.DeviceIdType.MESH,
            )

        xf = x_ref[...].reshape(B * SQ, D)
        qkv = jnp.dot(xf, wqkv_ref[...], preferred_element_type=jnp.float32)

        cos = cos_ref[...]
        sin = sin_ref[...]
        lane = lax.broadcasted_iota(jnp.int32, (B * SQ, DLOC), 1)
        even = (lane % 2) == 0

        def rope(t):
            rot = jnp.where(even, -jnp.roll(t, -1, axis=1),
                            jnp.roll(t, 1, axis=1))
            return t * cos + rot * sin

        q = rope(qkv[:, :DLOC].astype(bf16))
        k = rope(qkv[:, DLOC:2 * DLOC].astype(bf16))
        v = qkv[:, 2 * DLOC:]

        stage1 = [None] * n_chunk
        stage2 = [None] * n_chunk

        def finish_stage1(c):
            start, size = CHUNKS[c]
            sl = slice(start, start + size)
            stage1[c].wait()
            send_buf[sl] = send_buf[sl] + recv_buf[0, sl]
            rdma = exchange(1, c)
            rdma.start()
            stage2[c] = rdma

        for c in range(n_chunk):
            start, size = CHUNKS[c]
            rows = slice(start, start + size)
            batch = start // SQ
            krows = slice(batch * SQ, (batch + 1) * SQ)
            ctxs = []
            for h in range(HQ_LOCAL):
                cols = slice(h * DH, (h + 1) * DH)
                s = lax.dot_general(
                    q[rows, cols], k[krows, cols],
                    (((1,), (1,)), ((), ())),
                    preferred_element_type=jnp.float32)
                e = jnp.exp(s)
                denom = jnp.sum(e, axis=-1, keepdims=True)
                ctx = lax.dot_general(
                    e, v[krows, cols],
                    (((1,), (0,)), ((), ())),
                    preferred_element_type=jnp.float32) / denom
                ctxs.append(ctx)
            ctx16 = jnp.concatenate(ctxs, axis=1).astype(bf16)
            send_buf[rows] = jnp.dot(ctx16, wo_ref[...],
                                     preferred_element_type=jnp.float32
                                     ).astype(bf16)
            rdma = exchange(0, c)
            rdma.start()
            stage1[c] = rdma
            if c >= 2:
                finish_stage1(c - 2)

        finish_stage1(n_chunk - 2)
        finish_stage1(n_chunk - 1)

        for c in range(n_chunk):
            start, size = CHUNKS[c]
            sl = slice(start, start + size)
            stage2[c].wait()
            batch, rstart = divmod(start, SQ)
            out_ref[batch, rstart:rstart + size] = (
                send_buf[sl].astype(jnp.float32)
                + recv_buf[1, sl].astype(jnp.float32))

    cos, sin = _rope_tables()
    wqkv = jnp.concatenate([Wq * 0.125, Wk, Wv], axis=1).astype(bf16)
    args = (
        x.astype(bf16), wqkv, Wo.astype(bf16),
        jnp.asarray(cos, bf16), jnp.asarray(sin, bf16),
    )
    return pl.pallas_call(
        body,
        out_shape=jax.ShapeDtypeStruct((B, SQ, D), jnp.float32),
        in_specs=[pl.BlockSpec(memory_space=pltpu.VMEM)] * 5,
        out_specs=pl.BlockSpec(memory_space=pltpu.VMEM),
        scratch_shapes=[
            pltpu.VMEM((B * SQ, D), jnp.bfloat16),
            pltpu.VMEM((2, B * SQ, D), jnp.bfloat16),
            pltpu.SemaphoreType.DMA((2, len(CHUNKS))),
            pltpu.SemaphoreType.DMA((2, len(CHUNKS))),
        ],
        compiler_params=pltpu.CompilerParams(collective_id=0),
    )(*args)


# device time: 21377 ns/iter; 1.0964x vs baseline; 1.0964x over previous
import numpy as np
import jax
import jax.numpy as jnp
from jax import lax
from jax.experimental import pallas as pl
from jax.experimental.pallas import tpu as pltpu

N_DEV = 4
B, SQ, D = 2, 256, 768
HQ_LOCAL, DH = 4, 64
DLOC = HQ_LOCAL * DH
CHUNKS = [(0, 128), (128, 128), (256, 96), (352, 96), (448, 64)]


def _rope_tables():
    inv = 1.0 / (10000.0 ** (np.arange(0, DH, 2) / DH))
    pos = np.arange(SQ)[:, None] * inv[None, :]
    cos = np.repeat(np.cos(pos), 2, axis=-1).astype(np.float32)
    sin = np.repeat(np.sin(pos), 2, axis=-1).astype(np.float32)
    return np.tile(cos, (B, HQ_LOCAL)), np.tile(sin, (B, HQ_LOCAL))


def kernel(x, Wq, Wk, Wv, Wo):
    bf16 = jnp.bfloat16
    n_chunk = len(CHUNKS)

    def body(x_ref, wqkv_ref, wo_ref, cos_ref, sin_ref,
             out_ref, send_buf, recv_buf, send_sems, recv_sems):
        my_pos = lax.axis_index("i")
        p_a = my_pos ^ 1
        p_b = 3 - my_pos

        barrier_sem = pltpu.get_barrier_semaphore()
        for nbr in (p_a, p_b):
            pl.semaphore_signal(
                barrier_sem, inc=1,
                device_id=(nbr,), device_id_type=pl.DeviceIdType.MESH,
            )
        pl.semaphore_wait(barrier_sem, 2)

        def exchange(stage, c):
            start, size = CHUNKS[c]
            first, second = (p_a, p_b) if c % 2 == 0 else (p_b, p_a)
            partner = first if stage == 0 else second
            return pltpu.make_async_remote_copy(
                src_ref=send_buf.at[start:start + size],
                dst_ref=recv_buf.at[stage, start:start + size],
                send_sem=send_sems.at[stage, c],
                recv_sem=recv_sems.at[stage, c],
                device_id=(partner,),
                device_id_type=pl.DeviceIdType.MESH,
            )

        xf = x_ref[...].reshape(B * SQ, D)
        qkv = jnp.dot(xf, wqkv_ref[...], preferred_element_type=jnp.float32)

        cos = cos_ref[...]
        sin = sin_ref[...]
        lane = lax.broadcasted_iota(jnp.int32, (B * SQ, DLOC), 1)
        even = (lane % 2) == 0

        def rope(t):
            rot = jnp.where(even, -jnp.roll(t, -1, axis=1),
                            jnp.roll(t, 1, axis=1))
            return t * cos + rot * sin

        q = rope(qkv[:, :DLOC].astype(bf16))
        k = rope(qkv[:, DLOC:2 * DLOC].astype(bf16))
        v = qkv[:, 2 * DLOC:]

        stage1 = [None] * n_chunk
        stage2 = [None] * n_chunk

        def finish_stage1(c):
            start, size = CHUNKS[c]
            sl = slice(start, start + size)
            stage1[c].wait()
            send_buf[sl] = send_buf[sl] + recv_buf[0, sl]
            rdma = exchange(1, c)
            rdma.start()
            stage2[c] = rdma

        for c in range(n_chunk):
            start, size = CHUNKS[c]
            rows = slice(start, start + size)
            batch = start // SQ
            krows = slice(batch * SQ, (batch + 1) * SQ)
            ctxs = []
            for h in range(HQ_LOCAL):
                cols = slice(h * DH, (h + 1) * DH)
                s = lax.dot_general(
                    q[rows, cols], k[krows, cols],
                    (((1,), (1,)), ((), ())),
                    preferred_element_type=jnp.float32)
                e = jnp.exp(s)
                denom = jnp.sum(e, axis=-1, keepdims=True)
                ctx = lax.dot_general(
                    e, v[krows, cols],
                    (((1,), (0,)), ((), ())),
                    preferred_element_type=jnp.float32) / denom
                ctxs.append(ctx)
            ctx16 = jnp.concatenate(ctxs, axis=1).astype(bf16)
            send_buf[rows] = jnp.dot(ctx16, wo_ref[...],
                                     preferred_element_type=jnp.float32
                                     ).astype(bf16)
            rdma = exchange(0, c)
            rdma.start()
            stage1[c] = rdma

        for c in range(n_chunk):
            finish_stage1(c)

        for c in range(n_chunk):
            start, size = CHUNKS[c]
            sl = slice(start, start + size)
            stage2[c].wait()
            batch, rstart = divmod(start, SQ)
            out_ref[batch, rstart:rstart + size] = (
                send_buf[sl].astype(jnp.float32)
                + recv_buf[1, sl].astype(jnp.float32))

    cos, sin = _rope_tables()
    wqkv = jnp.concatenate([Wq * 0.125, Wk, Wv], axis=1).astype(bf16)
    args = (
        x.astype(bf16), wqkv, Wo.astype(bf16),
        jnp.asarray(cos, bf16), jnp.asarray(sin, bf16),
    )
    return pl.pallas_call(
        body,
        out_shape=jax.ShapeDtypeStruct((B, SQ, D), jnp.float32),
        in_specs=[pl.BlockSpec(memory_space=pltpu.VMEM)] * 5,
        out_specs=pl.BlockSpec(memory_space=pltpu.VMEM),
        scratch_shapes=[
            pltpu.VMEM((B * SQ, D), jnp.bfloat16),
            pltpu.VMEM((2, B * SQ, D), jnp.bfloat16),
            pltpu.SemaphoreType.DMA((2, len(CHUNKS))),
            pltpu.SemaphoreType.DMA((2, len(CHUNKS))),
        ],
        compiler_params=pltpu.CompilerParams(collective_id=0),
    )(*args)


# device time: 21225 ns/iter; 1.1042x vs baseline; 1.0072x over previous
import numpy as np
import jax
import jax.numpy as jnp
from jax import lax
from jax.experimental import pallas as pl
from jax.experimental.pallas import tpu as pltpu

N_DEV = 4
B, SQ, D = 2, 256, 768
HQ_LOCAL, DH = 4, 64
DLOC = HQ_LOCAL * DH
CHUNKS = [(0, 128), (128, 128), (256, 128), (384, 128)]


def _rope_tables():
    inv = 1.0 / (10000.0 ** (np.arange(0, DH, 2) / DH))
    pos = np.arange(SQ)[:, None] * inv[None, :]
    cos = np.repeat(np.cos(pos), 2, axis=-1).astype(np.float32)
    sin = np.repeat(np.sin(pos), 2, axis=-1).astype(np.float32)
    return np.tile(cos, (B, HQ_LOCAL)), np.tile(sin, (B, HQ_LOCAL))


def kernel(x, Wq, Wk, Wv, Wo):
    bf16 = jnp.bfloat16
    n_chunk = len(CHUNKS)

    def body(x_ref, wqkv_ref, wo_ref, cos_ref, sin_ref,
             out_ref, send_buf, recv_buf, send_sems, recv_sems):
        my_pos = lax.axis_index("i")
        p_a = my_pos ^ 1
        p_b = 3 - my_pos

        barrier_sem = pltpu.get_barrier_semaphore()
        for nbr in (p_a, p_b):
            pl.semaphore_signal(
                barrier_sem, inc=1,
                device_id=(nbr,), device_id_type=pl.DeviceIdType.MESH,
            )
        pl.semaphore_wait(barrier_sem, 2)

        def exchange(stage, c):
            start, size = CHUNKS[c]
            first, second = (p_a, p_b) if c % 2 == 0 else (p_b, p_a)
            partner = first if stage == 0 else second
            return pltpu.make_async_remote_copy(
                src_ref=send_buf.at[start:start + size],
                dst_ref=recv_buf.at[stage, start:start + size],
                send_sem=send_sems.at[stage, c],
                recv_sem=recv_sems.at[stage, c],
                device_id=(partner,),
                device_id_type=pl.DeviceIdType.MESH,
            )

        xf = x_ref[...].reshape(B * SQ, D)
        qkv = jnp.dot(xf, wqkv_ref[...], preferred_element_type=jnp.float32)

        cos = cos_ref[...]
        sin = sin_ref[...]
        lane = lax.broadcasted_iota(jnp.int32, (B * SQ, DLOC), 1)
        even = (lane % 2) == 0

        def rope(t):
            rot = jnp.where(even, -jnp.roll(t, -1, axis=1),
                            jnp.roll(t, 1, axis=1))
            return t * cos + rot * sin

        q = rope(qkv[:, :DLOC].astype(bf16))
        k = rope(qkv[:, DLOC:2 * DLOC].astype(bf16))
        v = qkv[:, 2 * DLOC:]

        stage1 = [None] * n_chunk
        stage2 = [None] * n_chunk

        def finish_stage1(c):
            start, size = CHUNKS[c]
            sl = slice(start, start + size)
            stage1[c].wait()
            send_buf[sl] = send_buf[sl] + recv_buf[0, sl]
            rdma = exchange(1, c)
            rdma.start()
            stage2[c] = rdma

        for c in range(n_chunk):
            start, size = CHUNKS[c]
            rows = slice(start, start + size)
            batch = start // SQ
            krows = slice(batch * SQ, (batch + 1) * SQ)
            ctxs = []
            for h in range(HQ_LOCAL):
                cols = slice(h * DH, (h + 1) * DH)
                s = lax.dot_general(
                    q[rows, cols], k[krows, cols],
                    (((1,), (1,)), ((), ())),
                    preferred_element_type=jnp.float32)
                e = jnp.exp(s)
                denom = jnp.sum(e, axis=-1, keepdims=True)
                ctx = lax.dot_general(
                    e, v[krows, cols],
                    (((1,), (0,)), ((), ())),
                    preferred_element_type=jnp.float32) / denom
                ctxs.append(ctx)
            ctx16 = jnp.concatenate(ctxs, axis=1).astype(bf16)
            send_buf[rows] = jnp.dot(ctx16, wo_ref[...],
                                     preferred_element_type=jnp.float32
                                     ).astype(bf16)
            rdma = exchange(0, c)
            rdma.start()
            stage1[c] = rdma

        for c in range(n_chunk):
            finish_stage1(c)

        for c in range(n_chunk):
            start, size = CHUNKS[c]
            sl = slice(start, start + size)
            stage2[c].wait()
            batch, rstart = divmod(start, SQ)
            out_ref[batch, rstart:rstart + size] = (
                send_buf[sl].astype(jnp.float32)
                + recv_buf[1, sl].astype(jnp.float32))

    cos, sin = _rope_tables()
    wqkv = jnp.concatenate([Wq * 0.125, Wk, Wv], axis=1).astype(bf16)
    args = (
        x.astype(bf16), wqkv, Wo.astype(bf16),
        jnp.asarray(cos, bf16), jnp.asarray(sin, bf16),
    )
    return pl.pallas_call(
        body,
        out_shape=jax.ShapeDtypeStruct((B, SQ, D), jnp.float32),
        in_specs=[pl.BlockSpec(memory_space=pltpu.VMEM)] * 5,
        out_specs=pl.BlockSpec(memory_space=pltpu.VMEM),
        scratch_shapes=[
            pltpu.VMEM((B * SQ, D), jnp.bfloat16),
            pltpu.VMEM((2, B * SQ, D), jnp.bfloat16),
            pltpu.SemaphoreType.DMA((2, len(CHUNKS))),
            pltpu.SemaphoreType.DMA((2, len(CHUNKS))),
        ],
        compiler_params=pltpu.CompilerParams(collective_id=0),
    )(*args)


# device time: 20231 ns/iter; 1.1585x vs baseline; 1.0491x over previous
import numpy as np
import jax
import jax.numpy as jnp
from jax import lax
from jax.experimental import pallas as pl
from jax.experimental.pallas import tpu as pltpu

N_DEV = 4
B, SQ, D = 2, 256, 768
HQ_LOCAL, DH = 4, 64
DLOC = HQ_LOCAL * DH
RQ = 128
N_CHUNK = B * SQ // RQ


def _rope_tables():
    inv = 1.0 / (10000.0 ** (np.arange(0, DH, 2) / DH))
    pos = np.arange(SQ)[:, None] * inv[None, :]
    cos = np.repeat(np.cos(pos), 2, axis=-1).astype(np.float32)
    sin = np.repeat(np.sin(pos), 2, axis=-1).astype(np.float32)
    return np.tile(cos, (B, HQ_LOCAL)), np.tile(sin, (B, HQ_LOCAL))


def kernel(x, Wq, Wk, Wv, Wo):
    bf16 = jnp.bfloat16

    def body(x_ref, wq_ref, wk_ref, wv_ref, wo_ref, cos_ref, sin_ref,
             out_ref, send_buf, recv_buf, send_sems, recv_sems):
        my_pos = lax.axis_index("i")
        p_a = my_pos ^ 1
        p_b = 3 - my_pos

        barrier_sem = pltpu.get_barrier_semaphore()
        for nbr in (p_a, p_b):
            pl.semaphore_signal(
                barrier_sem, inc=1,
                device_id=(nbr,), device_id_type=pl.DeviceIdType.MESH,
            )
        pl.semaphore_wait(barrier_sem, 2)

        def exchange(stage, chunk):
            first, second = (p_a, p_b) if chunk % 2 == 0 else (p_b, p_a)
            partner = first if stage == 0 else second
            return pltpu.make_async_remote_copy(
                src_ref=send_buf.at[chunk],
                dst_ref=recv_buf.at[stage, chunk],
                send_sem=send_sems.at[stage, chunk],
                recv_sem=recv_sems.at[stage, chunk],
                device_id=(partner,),
                device_id_type=pl.DeviceIdType.MESH,
            )

        xf = x_ref[...].reshape(B * SQ, D)
        q = jnp.dot(xf, wq_ref[...], preferred_element_type=jnp.float32)
        k = jnp.dot(xf, wk_ref[...], preferred_element_type=jnp.float32)
        v = jnp.dot(xf, wv_ref[...], preferred_element_type=jnp.float32)

        cos = cos_ref[...]
        sin = sin_ref[...]
        lane = lax.broadcasted_iota(jnp.int32, (B * SQ, DLOC), 1)
        even = (lane % 2) == 0

        def rope(t):
            rot = jnp.where(even, -jnp.roll(t, -1, axis=1),
                            jnp.roll(t, 1, axis=1))
            return t * cos + rot * sin

        q = rope(q.astype(bf16)) * jnp.asarray(0.125, bf16)
        k = rope(k.astype(bf16))

        stage1 = []
        for b in range(B):
            krows = slice(b * SQ, (b + 1) * SQ)
            ctxs = []
            for h in range(HQ_LOCAL):
                cols = slice(h * DH, (h + 1) * DH)
                s = lax.dot_general(
                    q[krows, cols], k[krows, cols],
                    (((1,), (1,)), ((), ())),
                    preferred_element_type=jnp.float32)
                e = jnp.exp(s)
                denom = jnp.sum(e, axis=-1, keepdims=True)
                ctx = lax.dot_general(
                    e, v[krows, cols],
                    (((1,), (0,)), ((), ())),
                    preferred_element_type=jnp.float32) / denom
                ctxs.append(ctx)
            ctx16 = jnp.concatenate(ctxs, axis=1).astype(bf16)
            for sub in range(2):
                chunk = 2 * b + sub
                send_buf[chunk] = jnp.dot(
                    ctx16[sub * RQ:(sub + 1) * RQ], wo_ref[...],
                    preferred_element_type=jnp.float32).astype(bf16)
                rdma = exchange(0, chunk)
                rdma.start()
                stage1.append(rdma)

        stage2 = []
        for chunk in range(N_CHUNK):
            stage1[chunk].wait()
            send_buf[chunk] = send_buf[chunk] + recv_buf[0, chunk]
            rdma = exchange(1, chunk)
            rdma.start()
            stage2.append(rdma)

        for chunk in range(N_CHUNK):
            stage2[chunk].wait()
            b, sub = divmod(chunk, 2)
            out_ref[b, sub * RQ:(sub + 1) * RQ] = (
                send_buf[chunk].astype(jnp.float32)
                + recv_buf[1, chunk].astype(jnp.float32))

    cos, sin = _rope_tables()
    args = (
        x.astype(bf16), Wq.astype(bf16), Wk.astype(bf16), Wv.astype(bf16),
        Wo.astype(bf16), jnp.asarray(cos, bf16), jnp.asarray(sin, bf16),
    )
    return pl.pallas_call(
        body,
        out_shape=jax.ShapeDtypeStruct((B, SQ, D), jnp.float32),
        in_specs=[pl.BlockSpec(memory_space=pltpu.VMEM)] * 7,
        out_specs=pl.BlockSpec(memory_space=pltpu.VMEM),
        scratch_shapes=[
            pltpu.VMEM((N_CHUNK, RQ, D), jnp.bfloat16),
            pltpu.VMEM((2, N_CHUNK, RQ, D), jnp.bfloat16),
            pltpu.SemaphoreType.DMA((2, N_CHUNK)),
            pltpu.SemaphoreType.DMA((2, N_CHUNK)),
        ],
        compiler_params=pltpu.CompilerParams(collective_id=0),
    )(*args)


# device time: 18768 ns/iter; 1.2488x vs baseline; 1.0780x over previous
import numpy as np
import jax
import jax.numpy as jnp
from jax import lax
from jax.experimental import pallas as pl
from jax.experimental.pallas import tpu as pltpu

N_DEV = 4
B, SQ, D = 2, 256, 768
HQ_LOCAL, DH = 4, 64
DLOC = HQ_LOCAL * DH
RQ = 64
N_CHUNK = B * SQ // RQ
SUBS = SQ // RQ


def _rope_tables():
    inv = 1.0 / (10000.0 ** (np.arange(0, DH, 2) / DH))
    pos = np.arange(SQ)[:, None] * inv[None, :]
    cos = np.repeat(np.cos(pos), 2, axis=-1).astype(np.float32)
    sin = np.repeat(np.sin(pos), 2, axis=-1).astype(np.float32)
    return np.tile(cos, (B, HQ_LOCAL)), np.tile(sin, (B, HQ_LOCAL))


def kernel(x, Wq, Wk, Wv, Wo):
    bf16 = jnp.bfloat16

    def body(x_ref, wq_ref, wk_ref, wv_ref, wo_ref, cos_ref, sin_ref,
             out_ref, send_buf, recv_buf, send_sems, recv_sems):
        my_pos = lax.axis_index("i")
        p_a = my_pos ^ 1
        p_b = 3 - my_pos

        barrier_sem = pltpu.get_barrier_semaphore()
        for nbr in (p_a, p_b):
            pl.semaphore_signal(
                barrier_sem, inc=1,
                device_id=(nbr,), device_id_type=pl.DeviceIdType.MESH,
            )
        pl.semaphore_wait(barrier_sem, 2)

        def exchange(stage, chunk):
            first, second = (p_a, p_b) if chunk % 2 == 0 else (p_b, p_a)
            partner = first if stage == 0 else second
            return pltpu.make_async_remote_copy(
                src_ref=send_buf.at[chunk],
                dst_ref=recv_buf.at[stage, chunk],
                send_sem=send_sems.at[stage, chunk],
                recv_sem=recv_sems.at[stage, chunk],
                device_id=(partner,),
                device_id_type=pl.DeviceIdType.MESH,
            )

        xf = x_ref[...].reshape(B * SQ, D)
        q = jnp.dot(xf, wq_ref[...], preferred_element_type=jnp.float32)
        k = jnp.dot(xf, wk_ref[...], preferred_element_type=jnp.float32)
        v = jnp.dot(xf, wv_ref[...], preferred_element_type=jnp.float32)

        cos = cos_ref[...]
        sin = sin_ref[...]
        lane = lax.broadcasted_iota(jnp.int32, (B * SQ, DLOC), 1)
        even = (lane % 2) == 0

        def rope(t):
            rot = jnp.where(even, -jnp.roll(t, -1, axis=1),
                            jnp.roll(t, 1, axis=1))
            return t * cos + rot * sin

        q = rope(q.astype(bf16)) * jnp.asarray(0.125, bf16)
        k = rope(k.astype(bf16))

        stage1 = []
        for b in range(B):
            krows = slice(b * SQ, (b + 1) * SQ)
            ctxs = []
            for h in range(HQ_LOCAL):
                cols = slice(h * DH, (h + 1) * DH)
                s = lax.dot_general(
                    q[krows, cols], k[krows, cols],
                    (((1,), (1,)), ((), ())),
                    preferred_element_type=jnp.float32)
                e = jnp.exp(s)
                denom = jnp.sum(e, axis=-1, keepdims=True)
                ctx = lax.dot_general(
                    e, v[krows, cols],
                    (((1,), (0,)), ((), ())),
                    preferred_element_type=jnp.float32) / denom
                ctxs.append(ctx)
            ctx16 = jnp.concatenate(ctxs, axis=1).astype(bf16)
            for sub in range(SUBS):
                chunk = SUBS * b + sub
                send_buf[chunk] = jnp.dot(
                    ctx16[sub * RQ:(sub + 1) * RQ], wo_ref[...],
                    preferred_element_type=jnp.float32).astype(bf16)
                rdma = exchange(0, chunk)
                rdma.start()
                stage1.append(rdma)

        stage2 = []
        for chunk in range(N_CHUNK):
            stage1[chunk].wait()
            send_buf[chunk] = send_buf[chunk] + recv_buf[0, chunk]
            rdma = exchange(1, chunk)
            rdma.start()
            stage2.append(rdma)

        for chunk in range(N_CHUNK):
            stage2[chunk].wait()
            b, sub = divmod(chunk, SUBS)
            out_ref[b, sub * RQ:(sub + 1) * RQ] = (
                send_buf[chunk].astype(jnp.float32)
                + recv_buf[1, chunk].astype(jnp.float32))

    cos, sin = _rope_tables()
    args = (
        x.astype(bf16), Wq.astype(bf16), Wk.astype(bf16), Wv.astype(bf16),
        Wo.astype(bf16), jnp.asarray(cos, bf16), jnp.asarray(sin, bf16),
    )
    return pl.pallas_call(
        body,
        out_shape=jax.ShapeDtypeStruct((B, SQ, D), jnp.float32),
        in_specs=[pl.BlockSpec(memory_space=pltpu.VMEM)] * 7,
        out_specs=pl.BlockSpec(memory_space=pltpu.VMEM),
        scratch_shapes=[
            pltpu.VMEM((N_CHUNK, RQ, D), jnp.bfloat16),
            pltpu.VMEM((2, N_CHUNK, RQ, D), jnp.bfloat16),
            pltpu.SemaphoreType.DMA((2, N_CHUNK)),
            pltpu.SemaphoreType.DMA((2, N_CHUNK)),
        ],
        compiler_params=pltpu.CompilerParams(collective_id=0),
    )(*args)
